# baseline (speedup 1.0000x reference)
import jax
import jax.numpy as jnp
from jax.experimental import pallas as pl
from jax.experimental.pallas import tpu as pltpu


def _cheb_fused_kernel(l_ref, x0_ref, wc_ref, b_ref, out_ref, x1_scr, x1b_scr):
    k = pl.program_id(0)
    r = pl.program_id(1)
    tile = out_ref.shape[0]
    f = x0_ref.shape[1]

    lb = l_ref[...].astype(jnp.bfloat16)

    @pl.when(k == 0)
    def _first_pass():
        x1 = jnp.dot(
            lb, x0_ref[...].astype(jnp.bfloat16), preferred_element_type=jnp.float32
        )
        x1_scr[pl.ds(r * tile, tile), :] = x1
        x1b_scr[pl.ds(r * tile, tile), :] = x1.astype(jnp.bfloat16)

    @pl.when(k == 1)
    def _second_pass():
        y = jnp.dot(lb, x1b_scr[...], preferred_element_type=jnp.float32)
        x0_r = x0_ref[pl.ds(r * tile, tile), :]
        x1_r = x1_scr[pl.ds(r * tile, tile), :]
        acc = jnp.dot(x0_r, wc_ref[0:f, :], preferred_element_type=jnp.float32)
        acc += jnp.dot(x1_r, wc_ref[f : 2 * f, :], preferred_element_type=jnp.float32)
        acc += jnp.dot(y, wc_ref[2 * f : 3 * f, :], preferred_element_type=jnp.float32)
        out_ref[...] = acc + b_ref[...]


def kernel(laplacian, inputs, weight, bias, precompute=0, einsum=0):
    B, Fin, V, X, Y, Z = inputs.shape
    K, _, Fout = weight.shape
    F = Fin * B * X * Y * Z

    x0 = jnp.transpose(inputs, (2, 1, 0, 3, 4, 5)).reshape(V, F)

    w0, w1, w2 = weight[0], weight[1], weight[2]
    wc = jnp.concatenate([w0 - w2, w1, 2.0 * w2], axis=0)
    b2d = bias.reshape(1, Fout)

    TILE = 512
    R = V // TILE

    out_flat = pl.pallas_call(
        _cheb_fused_kernel,
        grid=(2, R),
        in_specs=[
            pl.BlockSpec((TILE, V), lambda k, r: (r, 0)),
            pl.BlockSpec((V, F), lambda k, r: (0, 0)),
            pl.BlockSpec((3 * F, Fout), lambda k, r: (0, 0)),
            pl.BlockSpec((1, Fout), lambda k, r: (0, 0)),
        ],
        out_specs=pl.BlockSpec((TILE, Fout), lambda k, r: (r, 0)),
        out_shape=jax.ShapeDtypeStruct((V, Fout), jnp.float32),
        scratch_shapes=[
            pltpu.VMEM((V, F), jnp.float32),
            pltpu.VMEM((V, F), jnp.bfloat16),
        ],
    )(laplacian, x0, wc, b2d)

    out = jnp.transpose(out_flat, (1, 0)).reshape(B, Fout, V, X, Y, Z)
    return out


# speedup vs baseline: 1.0814x; 1.0814x over previous
import jax
import jax.numpy as jnp
from jax.experimental import pallas as pl
from jax.experimental.pallas import tpu as pltpu


def _cheb_fused_kernel(l_ref, x0t_ref, wc_ref, b_ref, out_ref, x0_scr, x1_scr):
    k = pl.program_id(0)
    r = pl.program_id(1)
    tile = l_ref.shape[0]
    f = x0t_ref.shape[0]

    @pl.when(jnp.logical_and(k == 0, r == 0))
    def _transpose_x0():
        x0_scr[...] = jnp.transpose(x0t_ref[...], (1, 0))

    @pl.when(k == 0)
    def _first_pass():
        x1_scr[pl.ds(r * tile, tile), :] = jnp.dot(
            l_ref[...], x0_scr[...], preferred_element_type=jnp.float32
        )

    @pl.when(k == 1)
    def _second_pass():
        y = jnp.dot(l_ref[...], x1_scr[...], preferred_element_type=jnp.float32)
        x0_r = x0_scr[pl.ds(r * tile, tile), :]
        x1_r = x1_scr[pl.ds(r * tile, tile), :]
        acc = jnp.dot(x0_r, wc_ref[0:f, :], preferred_element_type=jnp.float32)
        acc += jnp.dot(x1_r, wc_ref[f : 2 * f, :], preferred_element_type=jnp.float32)
        acc += jnp.dot(y, wc_ref[2 * f : 3 * f, :], preferred_element_type=jnp.float32)
        out_ref[...] = jnp.transpose(acc, (1, 0)) + b_ref[...]


def kernel(laplacian, inputs, weight, bias, precompute=0, einsum=0):
    B, Fin, V, X, Y, Z = inputs.shape
    K, _, Fout = weight.shape
    F = Fin * B * X * Y * Z

    x0t = inputs.reshape(F, V)

    w0, w1, w2 = weight[0], weight[1], weight[2]
    wc = jnp.concatenate([w0 - w2, w1, 2.0 * w2], axis=0)
    b2d = bias.reshape(Fout, 1)

    TILE = 512
    R = V // TILE

    out_t = pl.pallas_call(
        _cheb_fused_kernel,
        grid=(2, R),
        in_specs=[
            pl.BlockSpec((TILE, V), lambda k, r: (r, 0)),
            pl.BlockSpec((F, V), lambda k, r: (0, 0)),
            pl.BlockSpec((3 * F, Fout), lambda k, r: (0, 0)),
            pl.BlockSpec((Fout, 1), lambda k, r: (0, 0)),
        ],
        out_specs=pl.BlockSpec((Fout, TILE), lambda k, r: (0, r)),
        out_shape=jax.ShapeDtypeStruct((Fout, V), jnp.float32),
        scratch_shapes=[
            pltpu.VMEM((V, F), jnp.float32),
            pltpu.VMEM((V, F), jnp.float32),
        ],
    )(laplacian, x0t, wc, b2d)

    return out_t.reshape(B, Fout, V, X, Y, Z)
